# baseline (speedup 1.0000x reference)
import functools

import jax
import jax.numpy as jnp
from jax import lax
from jax.experimental import pallas as pl
from jax.experimental.pallas import tpu as pltpu
from jax.experimental.pallas import tpu_sc as plsc

_info = plsc.get_sparse_core_info()
_NC, _NS, _L = _info.num_cores, _info.num_subcores, _info.num_lanes
_NW = _NC * _NS
_R = 16
_U = 16


def _make_count_kernel(T, B, V):
    mesh = plsc.VectorSubcoreMesh(core_axis_name="c", subcore_axis_name="s")

    @functools.partial(
        pl.kernel,
        out_type=jax.ShapeDtypeStruct((_NW, _L), jnp.int32),
        mesh=mesh,
        compiler_params=pltpu.CompilerParams(needs_layout_passes=False),
        scratch_types=[
            pltpu.VMEM((_R, V), jnp.float32),
            pltpu.VMEM((_R, V), jnp.float32),
            pltpu.VMEM((B, T), jnp.int32),
            pltpu.VMEM((_L,), jnp.int32),
            pltpu.VMEM((_L,), jnp.int32),
            pltpu.VMEM((_L,), jnp.int32),
            pltpu.SemaphoreType.DMA,
            pltpu.SemaphoreType.DMA,
        ],
    )
    def count_kernel(outputs_hbm, tokens_hbm, t0s_hbm, cnts_hbm, out_hbm,
                     buf_a, buf_b, tok_v, t0_v, cnt_v, outb, sem_a, sem_b):
        c = lax.axis_index("c")
        s = lax.axis_index("s")
        wid = s * _NC + c
        pltpu.sync_copy(t0s_hbm.at[wid], t0_v)
        pltpu.sync_copy(cnts_hbm.at[wid], cnt_v)
        pltpu.sync_copy(tokens_hbm, tok_v)
        t0_vec = t0_v[...]
        cnt_vec = cnt_v[...]

        lanes = lax.iota(jnp.int32, _L)
        neg = jnp.full((_L,), -jnp.inf, jnp.float32)
        zero_i = jnp.zeros((_L,), jnp.int32)

        def xlane(vec, idx):
            return lax.gather(
                vec, idx[:, None],
                lax.GatherDimensionNumbers(
                    offset_dims=(), collapsed_slice_dims=(0,),
                    start_index_map=(0,)),
                (1,),
                mode=lax.GatherScatterMode.PROMISE_IN_BOUNDS)

        def bfly_max(v):
            for sh in (1, 2, 4, 8):
                v = jnp.maximum(v, xlane(v, lanes ^ sh))
            return v

        def bfly_min(v):
            for sh in (1, 2, 4, 8):
                v = jnp.minimum(v, xlane(v, lanes ^ sh))
            return v

        def rows_argmax2(buf, r):
            def chunk_body(jj, carry):
                (p0, p1, q0, q1, pj0, pj1, qj0, qj1) = carry
                pv = [p0, p1]
                qv = [q0, q1]
                pj = [pj0, pj1]
                qj = [qj0, qj1]
                for u in range(_U):
                    j = jj * _U + u
                    a = u % 2
                    x = buf[r, pl.ds(j * _L, _L)]
                    y = buf[r + 1, pl.ds(j * _L, _L)]
                    mx = x > pv[a]
                    my = y > qv[a]
                    pv[a] = jnp.where(mx, x, pv[a])
                    pj[a] = jnp.where(mx, j, pj[a])
                    qv[a] = jnp.where(my, y, qv[a])
                    qj[a] = jnp.where(my, j, qj[a])
                return (*pv, *qv, *pj, *qj)

            carry = (neg, neg, neg, neg, zero_i, zero_i, zero_i, zero_i)
            carry = lax.fori_loop(0, V // (_L * _U), chunk_body, carry)
            p0, p1, q0, q1, pj0, pj1, qj0, qj1 = carry

            def finish(v0, v1, j0, j1):
                take = (v1 > v0) | ((v1 == v0) & (j1 < j0))
                vm = jnp.where(take, v1, v0)
                jm = jnp.where(take, j1, j0)
                im = jm * _L + lanes
                gmax = bfly_max(vm)
                cand = jnp.where(vm == gmax, im, V)
                return bfly_min(cand)

            return finish(p0, p1, pj0, pj1), finish(q0, q1, qj0, qj1)

        acc = zero_i
        for b in range(B):
            t0 = t0_vec[b]
            cnt = cnt_vec[b]
            t1 = t0 + cnt
            nblk = (cnt + _R - 1) // _R
            npair = (nblk + 1) // 2

            def sclamp_of(k):
                return jnp.maximum(jnp.minimum(t0 + k * _R, t1 - _R), 0)

            def slice_of(k):
                return outputs_hbm.at[pl.ds(sclamp_of(k), _R), b, :]

            def compute_block(k, buf, acc):
                sraw = t0 + k * _R
                sclamp = sclamp_of(k)

                def row_body(h, acc):
                    r = h * 2
                    t = sclamp + r
                    g0, g1 = rows_argmax2(buf, r)
                    for d, g in ((1, g0), (2, g1)):
                        tpos = t + d
                        off = (tpos // _L) * _L
                        lane = tpos - off
                        chunk = tok_v[b, pl.ds(off, _L)]
                        hit = jnp.where((chunk == g) & (lanes == lane), 1, 0)
                        td = t + d - 1
                        valid = (td >= sraw) & (td < t1)
                        acc = acc + hit * jnp.where(valid, 1, 0)
                    return acc

                return lax.fori_loop(0, _R // 2, row_body, acc)

            @pl.when(nblk > 0)
            def _():
                pltpu.async_copy(slice_of(0), buf_a, sem_a)

            @pl.when(nblk > 1)
            def _():
                pltpu.async_copy(slice_of(1), buf_b, sem_b)

            def pair_body(i, acc):
                k0 = 2 * i
                k1 = k0 + 1
                pltpu.make_async_copy(slice_of(k0), buf_a, sem_a).wait()
                acc = compute_block(k0, buf_a, acc)

                @pl.when(k0 + 2 < nblk)
                def _():
                    pltpu.async_copy(slice_of(k0 + 2), buf_a, sem_a)

                @pl.when(k1 < nblk)
                def _():
                    pltpu.make_async_copy(slice_of(k1), buf_b, sem_b).wait()

                acc = compute_block(k1, buf_b, acc)

                @pl.when(k1 + 2 < nblk)
                def _():
                    pltpu.async_copy(slice_of(k1 + 2), buf_b, sem_b)

                return acc

            acc = lax.fori_loop(0, npair, pair_body, acc)

        outb[...] = acc
        pltpu.sync_copy(outb, out_hbm.at[wid])

    return count_kernel


@jax.jit
def kernel(outputs, tokens, tokens_lens):
    T, B, V = outputs.shape
    lens = (tokens_lens + 1).astype(jnp.int32)
    total = jnp.sum(lens)
    cum = jnp.concatenate(
        [jnp.zeros((1,), jnp.int32), jnp.cumsum(lens, dtype=jnp.int32)])
    w = jnp.arange(_NW, dtype=jnp.int32)
    lo = (w * total) // _NW
    hi = ((w + 1) * total) // _NW
    seg_lo = jnp.maximum(lo[:, None], cum[None, :-1])
    seg_hi = jnp.minimum(hi[:, None], cum[None, 1:])
    cnts = jnp.maximum(seg_hi - seg_lo, 0).astype(jnp.int32)
    t0s = jnp.maximum(seg_lo - cum[None, :-1], 0).astype(jnp.int32)
    pad = ((0, 0), (0, _L - B))
    t0s = jnp.pad(t0s, pad)
    cnts = jnp.pad(cnts, pad)
    tokens_t = tokens.T.astype(jnp.int32)

    counts = _make_count_kernel(T, B, V)(
        outputs, tokens_t, t0s, cnts)
    num = jnp.sum(counts).astype(jnp.float32)
    return num / total.astype(jnp.float32)


# speedup vs baseline: 1.2827x; 1.2827x over previous
import functools

import jax
import jax.numpy as jnp
from jax import lax
from jax.experimental import pallas as pl
from jax.experimental.pallas import tpu as pltpu
from jax.experimental.pallas import tpu_sc as plsc

_info = plsc.get_sparse_core_info()
_NC, _NS, _L = _info.num_cores, _info.num_subcores, _info.num_lanes
_NW = _NC * _NS
_R = 16
_U = 8


def _make_count_kernel(T, B, V):
    mesh = plsc.VectorSubcoreMesh(core_axis_name="c", subcore_axis_name="s")

    @functools.partial(
        pl.kernel,
        out_type=jax.ShapeDtypeStruct((_NW, _L), jnp.int32),
        mesh=mesh,
        compiler_params=pltpu.CompilerParams(needs_layout_passes=False),
        scratch_types=[
            pltpu.VMEM((_R, V), jnp.float32),
            pltpu.VMEM((_R, V), jnp.float32),
            pltpu.VMEM((B, T), jnp.int32),
            pltpu.VMEM((_L,), jnp.int32),
            pltpu.VMEM((_L,), jnp.int32),
            pltpu.VMEM((_L,), jnp.int32),
            pltpu.SemaphoreType.DMA,
            pltpu.SemaphoreType.DMA,
        ],
    )
    def count_kernel(outputs_hbm, tokens_hbm, t0s_hbm, cnts_hbm, out_hbm,
                     buf_a, buf_b, tok_v, t0_v, cnt_v, outb, sem_a, sem_b):
        c = lax.axis_index("c")
        s = lax.axis_index("s")
        wid = s * _NC + c
        pltpu.sync_copy(t0s_hbm.at[wid], t0_v)
        pltpu.sync_copy(cnts_hbm.at[wid], cnt_v)
        pltpu.sync_copy(tokens_hbm, tok_v)
        t0_vec = t0_v[...]
        cnt_vec = cnt_v[...]

        lanes = lax.iota(jnp.int32, _L)
        neg = jnp.full((_L,), -jnp.inf, jnp.float32)
        zero_i = jnp.zeros((_L,), jnp.int32)

        def xlane(vec, idx):
            return lax.gather(
                vec, idx[:, None],
                lax.GatherDimensionNumbers(
                    offset_dims=(), collapsed_slice_dims=(0,),
                    start_index_map=(0,)),
                (1,),
                mode=lax.GatherScatterMode.PROMISE_IN_BOUNDS)

        def bfly_max(v):
            for sh in (1, 2, 4, 8):
                v = jnp.maximum(v, xlane(v, lanes ^ sh))
            return v

        def bfly_min(v):
            for sh in (1, 2, 4, 8):
                v = jnp.minimum(v, xlane(v, lanes ^ sh))
            return v

        def rows_argmax2(buf, r):
            def chunk_body(jj, carry):
                (p0, p1, q0, q1, pj0, pj1, qj0, qj1) = carry
                pv = [p0, p1]
                qv = [q0, q1]
                pj = [pj0, pj1]
                qj = [qj0, qj1]
                for u in range(_U):
                    j = jj * _U + u
                    a = u % 2
                    x = buf[r, pl.ds(j * _L, _L)]
                    y = buf[r + 1, pl.ds(j * _L, _L)]
                    mx = x > pv[a]
                    my = y > qv[a]
                    pv[a] = jnp.where(mx, x, pv[a])
                    pj[a] = jnp.where(mx, j, pj[a])
                    qv[a] = jnp.where(my, y, qv[a])
                    qj[a] = jnp.where(my, j, qj[a])
                return (*pv, *qv, *pj, *qj)

            carry = (neg, neg, neg, neg, zero_i, zero_i, zero_i, zero_i)
            carry = lax.fori_loop(0, V // (_L * _U), chunk_body, carry)
            p0, p1, q0, q1, pj0, pj1, qj0, qj1 = carry

            def finish(v0, v1, j0, j1):
                take = (v1 > v0) | ((v1 == v0) & (j1 < j0))
                vm = jnp.where(take, v1, v0)
                jm = jnp.where(take, j1, j0)
                im = jm * _L + lanes
                gmax = bfly_max(vm)
                cand = jnp.where(vm == gmax, im, V)
                return bfly_min(cand)

            return finish(p0, p1, pj0, pj1), finish(q0, q1, qj0, qj1)

        acc = zero_i
        for b in range(B):
            t0 = t0_vec[b]
            cnt = cnt_vec[b]
            t1 = t0 + cnt
            nblk = (cnt + _R - 1) // _R
            npair = (nblk + 1) // 2

            def sclamp_of(k):
                return jnp.maximum(jnp.minimum(t0 + k * _R, t1 - _R), 0)

            def slice_of(k):
                return outputs_hbm.at[pl.ds(sclamp_of(k), _R), b, :]

            def compute_block(k, buf, acc):
                sraw = t0 + k * _R
                sclamp = sclamp_of(k)

                def row_body(h, acc):
                    r = h * 2
                    t = sclamp + r
                    g0, g1 = rows_argmax2(buf, r)
                    for d, g in ((1, g0), (2, g1)):
                        tpos = t + d
                        off = (tpos // _L) * _L
                        lane = tpos - off
                        chunk = tok_v[b, pl.ds(off, _L)]
                        hit = jnp.where((chunk == g) & (lanes == lane), 1, 0)
                        td = t + d - 1
                        valid = (td >= sraw) & (td < t1)
                        acc = acc + hit * jnp.where(valid, 1, 0)
                    return acc

                return lax.fori_loop(0, _R // 2, row_body, acc)

            @pl.when(nblk > 0)
            def _():
                pltpu.async_copy(slice_of(0), buf_a, sem_a)

            @pl.when(nblk > 1)
            def _():
                pltpu.async_copy(slice_of(1), buf_b, sem_b)

            def pair_body(i, acc):
                k0 = 2 * i
                k1 = k0 + 1
                pltpu.make_async_copy(slice_of(k0), buf_a, sem_a).wait()
                acc = compute_block(k0, buf_a, acc)

                @pl.when(k0 + 2 < nblk)
                def _():
                    pltpu.async_copy(slice_of(k0 + 2), buf_a, sem_a)

                @pl.when(k1 < nblk)
                def _():
                    pltpu.make_async_copy(slice_of(k1), buf_b, sem_b).wait()

                acc = compute_block(k1, buf_b, acc)

                @pl.when(k1 + 2 < nblk)
                def _():
                    pltpu.async_copy(slice_of(k1 + 2), buf_b, sem_b)

                return acc

            acc = lax.fori_loop(0, npair, pair_body, acc)

        outb[...] = acc
        pltpu.sync_copy(outb, out_hbm.at[wid])

    return count_kernel


@jax.jit
def kernel(outputs, tokens, tokens_lens):
    T, B, V = outputs.shape
    lens = (tokens_lens + 1).astype(jnp.int32)
    total = jnp.sum(lens)
    cum = jnp.concatenate(
        [jnp.zeros((1,), jnp.int32), jnp.cumsum(lens, dtype=jnp.int32)])
    w = jnp.arange(_NW, dtype=jnp.int32)
    lo = (w * total) // _NW
    hi = ((w + 1) * total) // _NW
    seg_lo = jnp.maximum(lo[:, None], cum[None, :-1])
    seg_hi = jnp.minimum(hi[:, None], cum[None, 1:])
    cnts = jnp.maximum(seg_hi - seg_lo, 0).astype(jnp.int32)
    t0s = jnp.maximum(seg_lo - cum[None, :-1], 0).astype(jnp.int32)
    pad = ((0, 0), (0, _L - B))
    t0s = jnp.pad(t0s, pad)
    cnts = jnp.pad(cnts, pad)
    tokens_t = tokens.T.astype(jnp.int32)

    counts = _make_count_kernel(T, B, V)(
        outputs, tokens_t, t0s, cnts)
    num = jnp.sum(counts).astype(jnp.float32)
    return num / total.astype(jnp.float32)
